# baseline (speedup 1.0000x reference)
import jax
import jax.numpy as jnp
from jax.experimental import pallas as pl


def _cd_row_kernel(s_ref, t_ref, out_ref):
    s = s_ref[0]
    for j in range(16):
        t = t_ref[j]
        d2 = jax.lax.dot_general(
            s, t, (((1,), (1,)), ((), ())), preferred_element_type=jnp.float32
        )
        rowmin = jnp.min(d2, axis=1)
        colmin = jnp.min(d2, axis=0)
        fwd = jnp.mean(jnp.sqrt(jnp.maximum(rowmin, 1e-12)))
        bwd = jnp.mean(jnp.sqrt(jnp.maximum(colmin, 1e-12)))
        out_ref[0, j] = jnp.full((1, 128), fwd + bwd, dtype=jnp.float32)


def _hi_lo(x):
    hi = x.astype(jnp.bfloat16).astype(jnp.float32)
    return hi, x - hi


@jax.jit
def kernel(source, target):
    b, n, _ = source.shape
    ns = jnp.sum(source * source, axis=-1, keepdims=True)
    nt = jnp.sum(target * target, axis=-1, keepdims=True)
    ns_hi, ns_lo = _hi_lo(ns)
    nt_hi, nt_lo = _hi_lo(nt)
    s_r = source.astype(jnp.bfloat16).astype(jnp.float32)
    t_r = target.astype(jnp.bfloat16).astype(jnp.float32)
    ones = jnp.ones_like(ns)
    zero = jnp.zeros_like(ns)
    s_aug = jnp.concatenate([-2.0 * s_r, ns_hi, ns_lo, ones, ones, zero], axis=-1)
    t_aug = jnp.concatenate([t_r, ones, ones, nt_hi, nt_lo, zero], axis=-1)
    s_aug = s_aug.astype(jnp.bfloat16)
    t_aug = t_aug.astype(jnp.bfloat16)

    cd = pl.pallas_call(
        _cd_row_kernel,
        grid=(b,),
        in_specs=[
            pl.BlockSpec((1, n, 8), lambda i: (i, 0, 0)),
            pl.BlockSpec((b, n, 8), lambda i: (0, 0, 0)),
        ],
        out_specs=pl.BlockSpec((1, b, 1, 128), lambda i: (i, 0, 0, 0)),
        out_shape=jax.ShapeDtypeStruct((b, b, 1, 128), jnp.float32),
    )(s_aug, t_aug)[:, :, 0, 0]

    return jnp.mean(jnp.min(cd, axis=1))


# speedup vs baseline: 1.1963x; 1.1963x over previous
import jax
import jax.numpy as jnp
from jax.experimental import pallas as pl


def _cd_row_kernel(s_ref, t_ref, out_ref):
    s = s_ref[0]
    for j in range(4):
        t = t_ref[j]
        d2 = jax.lax.dot_general(
            s, t, (((1,), (1,)), ((), ())), preferred_element_type=jnp.float32
        )
        rowmin = jnp.min(d2, axis=1)
        colmin = jnp.min(d2, axis=0)
        fwd = jnp.mean(jnp.sqrt(jnp.maximum(rowmin, 1e-12)))
        bwd = jnp.mean(jnp.sqrt(jnp.maximum(colmin, 1e-12)))
        out_ref[0, j] = jnp.full((1, 128), fwd + bwd, dtype=jnp.float32)


def _hi_lo(x):
    hi = x.astype(jnp.bfloat16).astype(jnp.float32)
    return hi, x - hi


@jax.jit
def kernel(source, target):
    b, n, _ = source.shape
    ns = jnp.sum(source * source, axis=-1, keepdims=True)
    nt = jnp.sum(target * target, axis=-1, keepdims=True)
    ns_hi, ns_lo = _hi_lo(ns)
    nt_hi, nt_lo = _hi_lo(nt)
    s_r = source.astype(jnp.bfloat16).astype(jnp.float32)
    t_r = target.astype(jnp.bfloat16).astype(jnp.float32)
    ones = jnp.ones_like(ns)
    zero = jnp.zeros_like(ns)
    s_aug = jnp.concatenate([-2.0 * s_r, ns_hi, ns_lo, ones, ones, zero], axis=-1)
    t_aug = jnp.concatenate([t_r, ones, ones, nt_hi, nt_lo, zero], axis=-1)
    s_aug = s_aug.astype(jnp.bfloat16)
    t_aug = t_aug.astype(jnp.bfloat16)

    cd = pl.pallas_call(
        _cd_row_kernel,
        grid=(b, b // 4),
        in_specs=[
            pl.BlockSpec((1, n, 8), lambda i, jb: (i, 0, 0)),
            pl.BlockSpec((4, n, 8), lambda i, jb: (jb, 0, 0)),
        ],
        out_specs=pl.BlockSpec((1, 4, 1, 128), lambda i, jb: (i, jb, 0, 0)),
        out_shape=jax.ShapeDtypeStruct((b, b, 1, 128), jnp.float32),
    )(s_aug, t_aug)[:, :, 0, 0]

    return jnp.mean(jnp.min(cd, axis=1))


# speedup vs baseline: 1.2207x; 1.0204x over previous
import jax
import jax.numpy as jnp
from jax.experimental import pallas as pl


def _cd_row_kernel(s_ref, t_ref, out_ref):
    s = s_ref[0]
    for j in range(8):
        t = t_ref[j]
        d2 = jax.lax.dot_general(
            s, t, (((1,), (1,)), ((), ())), preferred_element_type=jnp.float32
        )
        rowmin = jnp.min(d2, axis=1)
        colmin = jnp.min(d2, axis=0)
        fwd = jnp.mean(jnp.sqrt(jnp.maximum(rowmin, 1e-12)))
        bwd = jnp.mean(jnp.sqrt(jnp.maximum(colmin, 1e-12)))
        out_ref[0, j] = jnp.full((1, 128), fwd + bwd, dtype=jnp.float32)


def _hi_lo(x):
    hi = x.astype(jnp.bfloat16).astype(jnp.float32)
    return hi, x - hi


@jax.jit
def kernel(source, target):
    b, n, _ = source.shape
    ns = jnp.sum(source * source, axis=-1, keepdims=True)
    nt = jnp.sum(target * target, axis=-1, keepdims=True)
    ns_hi, ns_lo = _hi_lo(ns)
    nt_hi, nt_lo = _hi_lo(nt)
    s_r = source.astype(jnp.bfloat16).astype(jnp.float32)
    t_r = target.astype(jnp.bfloat16).astype(jnp.float32)
    ones = jnp.ones_like(ns)
    zero = jnp.zeros_like(ns)
    s_aug = jnp.concatenate([-2.0 * s_r, ns_hi, ns_lo, ones, ones, zero], axis=-1)
    t_aug = jnp.concatenate([t_r, ones, ones, nt_hi, nt_lo, zero], axis=-1)
    s_aug = s_aug.astype(jnp.bfloat16)
    t_aug = t_aug.astype(jnp.bfloat16)

    cd = pl.pallas_call(
        _cd_row_kernel,
        grid=(b, b // 8),
        in_specs=[
            pl.BlockSpec((1, n, 8), lambda i, jb: (i, 0, 0)),
            pl.BlockSpec((8, n, 8), lambda i, jb: (jb, 0, 0)),
        ],
        out_specs=pl.BlockSpec((1, 8, 1, 128), lambda i, jb: (i, jb, 0, 0)),
        out_shape=jax.ShapeDtypeStruct((b, b, 1, 128), jnp.float32),
    )(s_aug, t_aug)[:, :, 0, 0]

    return jnp.mean(jnp.min(cd, axis=1))


# speedup vs baseline: 1.2392x; 1.0151x over previous
import jax
import jax.numpy as jnp
from jax.experimental import pallas as pl


def _cd_row_kernel(s_ref, t_ref, out_ref):
    s = s_ref[0]
    for j in range(8):
        t = t_ref[j]
        fwd_sum = 0.0
        colmin = None
        for r in range(4):
            d2 = jax.lax.dot_general(
                s[r * 512:(r + 1) * 512], t, (((1,), (1,)), ((), ())),
                preferred_element_type=jnp.float32,
            )
            rowmin = jnp.min(d2, axis=1)
            cmin = jnp.min(d2, axis=0)
            colmin = cmin if colmin is None else jnp.minimum(colmin, cmin)
            fwd_sum += jnp.sum(jnp.sqrt(jnp.maximum(rowmin, 1e-12)))
        fwd = fwd_sum / 2048.0
        bwd = jnp.mean(jnp.sqrt(jnp.maximum(colmin, 1e-12)))
        out_ref[0, j] = jnp.full((1, 128), fwd + bwd, dtype=jnp.float32)


def _hi_lo(x):
    hi = x.astype(jnp.bfloat16).astype(jnp.float32)
    return hi, x - hi


@jax.jit
def kernel(source, target):
    b, n, _ = source.shape
    ns = jnp.sum(source * source, axis=-1, keepdims=True)
    nt = jnp.sum(target * target, axis=-1, keepdims=True)
    ns_hi, ns_lo = _hi_lo(ns)
    nt_hi, nt_lo = _hi_lo(nt)
    s_r = source.astype(jnp.bfloat16).astype(jnp.float32)
    t_r = target.astype(jnp.bfloat16).astype(jnp.float32)
    ones = jnp.ones_like(ns)
    zero = jnp.zeros_like(ns)
    s_aug = jnp.concatenate([-2.0 * s_r, ns_hi, ns_lo, ones, ones, zero], axis=-1)
    t_aug = jnp.concatenate([t_r, ones, ones, nt_hi, nt_lo, zero], axis=-1)
    s_aug = s_aug.astype(jnp.bfloat16)
    t_aug = t_aug.astype(jnp.bfloat16)

    cd = pl.pallas_call(
        _cd_row_kernel,
        grid=(b, b // 8),
        in_specs=[
            pl.BlockSpec((1, n, 8), lambda i, jb: (i, 0, 0)),
            pl.BlockSpec((8, n, 8), lambda i, jb: (jb, 0, 0)),
        ],
        out_specs=pl.BlockSpec((1, 8, 1, 128), lambda i, jb: (i, jb, 0, 0)),
        out_shape=jax.ShapeDtypeStruct((b, b, 1, 128), jnp.float32),
    )(s_aug, t_aug)[:, :, 0, 0]

    return jnp.mean(jnp.min(cd, axis=1))


# speedup vs baseline: 1.4915x; 1.2036x over previous
import jax
import jax.numpy as jnp
from jax.experimental import pallas as pl


def _augment_kernel(s_ref, t_ref, sa_ref, ta_ref):
    for c in range(16):
        s = s_ref[c]
        t = t_ref[c]
        ns = jnp.sum(s * s, axis=1, keepdims=True)
        nt = jnp.sum(t * t, axis=1, keepdims=True)
        ns_hi = ns.astype(jnp.bfloat16).astype(jnp.float32)
        ns_lo = ns - ns_hi
        nt_hi = nt.astype(jnp.bfloat16).astype(jnp.float32)
        nt_lo = nt - nt_hi
        ones = jnp.ones_like(ns)
        sa = jnp.concatenate([-2.0 * s, ns_hi, ns_lo, ones, ones, ns * 0.0], axis=1)
        ta = jnp.concatenate([t, ones, ones, nt_hi, nt_lo, nt * 0.0], axis=1)
        sa_ref[c] = sa.astype(jnp.bfloat16)
        ta_ref[c] = ta.astype(jnp.bfloat16)


def _cd_row_kernel(s_ref, t_ref, out_ref):
    s = s_ref[0]
    for j in range(8):
        t = t_ref[j]
        fwd_sum = 0.0
        colmin = None
        for r in range(4):
            d2 = jax.lax.dot_general(
                s[r * 512:(r + 1) * 512], t, (((1,), (1,)), ((), ())),
                preferred_element_type=jnp.float32,
            )
            rowmin = jnp.min(d2, axis=1)
            cmin = jnp.min(d2, axis=0)
            colmin = cmin if colmin is None else jnp.minimum(colmin, cmin)
            fwd_sum += jnp.sum(jnp.sqrt(jnp.maximum(rowmin, 1e-12)))
        fwd = fwd_sum / 2048.0
        bwd = jnp.mean(jnp.sqrt(jnp.maximum(colmin, 1e-12)))
        out_ref[0, j] = jnp.full((1, 128), fwd + bwd, dtype=jnp.float32)


@jax.jit
def kernel(source, target):
    b, n, _ = source.shape
    s_aug, t_aug = pl.pallas_call(
        _augment_kernel,
        out_shape=[
            jax.ShapeDtypeStruct((b, n, 8), jnp.bfloat16),
            jax.ShapeDtypeStruct((b, n, 8), jnp.bfloat16),
        ],
    )(source, target)

    cd = pl.pallas_call(
        _cd_row_kernel,
        grid=(b, b // 8),
        in_specs=[
            pl.BlockSpec((1, n, 8), lambda i, jb: (i, 0, 0)),
            pl.BlockSpec((8, n, 8), lambda i, jb: (jb, 0, 0)),
        ],
        out_specs=pl.BlockSpec((1, 8, 1, 128), lambda i, jb: (i, jb, 0, 0)),
        out_shape=jax.ShapeDtypeStruct((b, b, 1, 128), jnp.float32),
    )(s_aug, t_aug)[:, :, 0, 0]

    return jnp.mean(jnp.min(cd, axis=1))
